# baseline (speedup 1.0000x reference)
import functools

import jax
import jax.numpy as jnp
from jax import lax
from jax.experimental import pallas as pl
from jax.experimental.pallas import tpu as pltpu

_BF = jnp.bfloat16
_F32 = jnp.float32


def _l2norm(h):
    sumsq = jnp.sum(h * h, axis=-1, keepdims=True)
    return h * lax.rsqrt(jnp.maximum(sumsq, 1e-24))


def _pre_kernel(x_ref, wpre_ref, bpre_ref, wmp0_ref, t_ref):
    x = x_ref[...].astype(_BF)
    h = jnp.dot(x, wpre_ref[...], preferred_element_type=_F32) + bpre_ref[...]
    h = jnp.maximum(h, 0.0).astype(_BF)
    t = jnp.dot(h, wmp0_ref[...], preferred_element_type=_F32)
    t_ref[...] = t.astype(_BF)


def _layer0_kernel(a_ref, t_ref, bmp0_ref, wmp1_ref, u_ref):
    a = a_ref[...].astype(_BF)
    h = jnp.dot(a, t_ref[...], preferred_element_type=_F32) + bmp0_ref[...]
    h = _l2norm(jnp.maximum(h, 0.0)).astype(_BF)
    u_ref[...] = jnp.dot(h, wmp1_ref[...], preferred_element_type=_F32).astype(_BF)


def _layer1_kernel(num_snapshots, dim_inner,
                   a_ref, u_ref, bsum_ref, whead_ref, bhead_ref, o_ref):
    acc = jnp.dot(a_ref[0].astype(_BF), u_ref[:, 0:dim_inner],
                  preferred_element_type=_F32)
    for s in range(1, num_snapshots):
        acc = acc + jnp.dot(a_ref[s].astype(_BF),
                            u_ref[:, s * dim_inner:(s + 1) * dim_inner],
                            preferred_element_type=_F32)
    h = _l2norm(jnp.maximum(acc + bsum_ref[...], 0.0)).astype(_BF)
    out = jnp.dot(h, whead_ref[...], preferred_element_type=_F32) + bhead_ref[...]
    o_ref[...] = out.astype(o_ref.dtype)


def kernel(x, adj_full, adj_snapshots,
           w_pre, b_pre, w_mp0, b_mp0, w_mp1, b_mp1, w_head, b_head):
    N, dim_in = x.shape
    S = adj_snapshots.shape[0]
    dim_inner = w_pre.shape[1]
    dim_out = w_head.shape[1]

    w_mp1_flat = jnp.transpose(w_mp1, (1, 0, 2)).reshape(
        dim_inner, S * dim_inner).astype(_BF)
    wpre16 = w_pre.astype(_BF)
    wmp0_16 = w_mp0.astype(_BF)
    whead16 = w_head.astype(_BF)
    bpre = b_pre.reshape(1, dim_inner)
    bmp0 = b_mp0.reshape(1, dim_inner)
    bsum = jnp.sum(b_mp1, axis=0).reshape(1, dim_inner)
    bhead = b_head.reshape(1, dim_out)

    psem = pltpu.CompilerParams(dimension_semantics=("parallel",))

    blk_pre = N if N < 512 else N // 2
    t = pl.pallas_call(
        _pre_kernel,
        out_shape=jax.ShapeDtypeStruct((N, dim_inner), _BF),
        grid=(N // blk_pre,),
        in_specs=[
            pl.BlockSpec((blk_pre, dim_in), lambda i: (i, 0)),
            pl.BlockSpec((dim_in, dim_inner), lambda i: (0, 0)),
            pl.BlockSpec((1, dim_inner), lambda i: (0, 0)),
            pl.BlockSpec((dim_inner, dim_inner), lambda i: (0, 0)),
        ],
        out_specs=pl.BlockSpec((blk_pre, dim_inner), lambda i: (i, 0)),
        compiler_params=psem,
    )(x, wpre16, bpre, wmp0_16)

    blk = 256 if N % 256 == 0 else N
    u = pl.pallas_call(
        _layer0_kernel,
        out_shape=jax.ShapeDtypeStruct((N, S * dim_inner), _BF),
        grid=(N // blk,),
        in_specs=[
            pl.BlockSpec((blk, N), lambda i: (i, 0)),
            pl.BlockSpec((N, dim_inner), lambda i: (0, 0)),
            pl.BlockSpec((1, dim_inner), lambda i: (0, 0)),
            pl.BlockSpec((dim_inner, S * dim_inner), lambda i: (0, 0)),
        ],
        out_specs=pl.BlockSpec((blk, S * dim_inner), lambda i: (i, 0)),
        compiler_params=psem,
    )(adj_full, t, bmp0, w_mp1_flat)

    return pl.pallas_call(
        functools.partial(_layer1_kernel, S, dim_inner),
        out_shape=jax.ShapeDtypeStruct((N, dim_out), x.dtype),
        grid=(N // blk,),
        in_specs=[
            pl.BlockSpec((S, blk, N), lambda i: (0, i, 0)),
            pl.BlockSpec((N, S * dim_inner), lambda i: (0, 0)),
            pl.BlockSpec((1, dim_inner), lambda i: (0, 0)),
            pl.BlockSpec((dim_inner, dim_out), lambda i: (0, 0)),
            pl.BlockSpec((1, dim_out), lambda i: (0, 0)),
        ],
        out_specs=pl.BlockSpec((blk, dim_out), lambda i: (i, 0)),
        compiler_params=psem,
    )(adj_snapshots, u, bsum, whead16, bhead)


# speedup vs baseline: 1.4262x; 1.4262x over previous
import functools

import jax
import jax.numpy as jnp
from jax import lax
from jax.experimental import pallas as pl
from jax.experimental.pallas import tpu as pltpu

_F32 = jnp.float32


def _l2norm(h):
    sumsq = jnp.sum(h * h, axis=-1, keepdims=True)
    return h * lax.rsqrt(jnp.maximum(sumsq, 1e-24))


def _fused_kernel(nb, num_snapshots, dim_inner,
                  x_ref, af_ref, as_ref,
                  wpre_ref, bpre_ref, wmp0_ref, bmp0_ref,
                  wmp1_ref, bsum_ref, whead_ref, bhead_ref,
                  o_ref, t_ref, acc_ref, w1_ref):
    i = pl.program_id(0)
    S, D = num_snapshots, dim_inner

    @pl.when(i == 0)
    def _init():
        h = jnp.dot(x_ref[...], wpre_ref[...],
                    preferred_element_type=_F32) + bpre_ref[...]
        h = jnp.maximum(h, 0.0)
        t_ref[...] = jnp.dot(h, wmp0_ref[...], preferred_element_type=_F32)
        w1_ref[...] = jnp.concatenate(
            [wmp1_ref[s] for s in range(S)], axis=1)
        acc_ref[...] = jnp.zeros_like(acc_ref)

    h1 = jnp.dot(af_ref[...], t_ref[...],
                 preferred_element_type=_F32) + bmp0_ref[...]
    h1 = _l2norm(jnp.maximum(h1, 0.0))

    u = jnp.dot(h1, w1_ref[...], preferred_element_type=_F32)
    partial = jnp.dot(as_ref[0], u[:, 0:D], preferred_element_type=_F32)
    for s in range(1, S):
        partial = partial + jnp.dot(as_ref[s], u[:, s * D:(s + 1) * D],
                                    preferred_element_type=_F32)
    acc_ref[...] += partial

    @pl.when(i == nb - 1)
    def _finish():
        hf = _l2norm(jnp.maximum(acc_ref[...] + bsum_ref[...], 0.0))
        out = jnp.dot(hf, whead_ref[...],
                      preferred_element_type=_F32) + bhead_ref[...]
        o_ref[...] = out.astype(o_ref.dtype)


def kernel(x, adj_full, adj_snapshots,
           w_pre, b_pre, w_mp0, b_mp0, w_mp1, b_mp1, w_head, b_head):
    N, dim_in = x.shape
    S = adj_snapshots.shape[0]
    dim_inner = w_pre.shape[1]
    dim_out = w_head.shape[1]

    blk = 128 if N % 128 == 0 and N > 128 else N
    nb = N // blk

    bpre = b_pre.reshape(1, dim_inner)
    bmp0 = b_mp0.reshape(1, dim_inner)
    bsum = jnp.sum(b_mp1, axis=0).reshape(1, dim_inner)
    bhead = b_head.reshape(1, dim_out)

    return pl.pallas_call(
        functools.partial(_fused_kernel, nb, S, dim_inner),
        out_shape=jax.ShapeDtypeStruct((N, dim_out), x.dtype),
        grid=(nb,),
        in_specs=[
            pl.BlockSpec((N, dim_in), lambda i: (0, 0)),
            pl.BlockSpec((blk, N), lambda i: (i, 0)),
            pl.BlockSpec((S, N, blk), lambda i: (0, 0, i)),
            pl.BlockSpec((dim_in, dim_inner), lambda i: (0, 0)),
            pl.BlockSpec((1, dim_inner), lambda i: (0, 0)),
            pl.BlockSpec((dim_inner, dim_inner), lambda i: (0, 0)),
            pl.BlockSpec((1, dim_inner), lambda i: (0, 0)),
            pl.BlockSpec((S, dim_inner, dim_inner), lambda i: (0, 0, 0)),
            pl.BlockSpec((1, dim_inner), lambda i: (0, 0)),
            pl.BlockSpec((dim_inner, dim_out), lambda i: (0, 0)),
            pl.BlockSpec((1, dim_out), lambda i: (0, 0)),
        ],
        out_specs=pl.BlockSpec((N, dim_out), lambda i: (0, 0)),
        scratch_shapes=[
            pltpu.VMEM((N, dim_inner), _F32),
            pltpu.VMEM((N, dim_inner), _F32),
            pltpu.VMEM((dim_inner, S * dim_inner), _F32),
        ],
        compiler_params=pltpu.CompilerParams(
            dimension_semantics=("arbitrary",)),
    )(x, adj_full, adj_snapshots,
      w_pre, bpre, w_mp0, bmp0, w_mp1, bsum, w_head, bhead)
